# baseline (speedup 1.0000x reference)
import jax
import jax.numpy as jnp
from jax import lax
from jax.experimental import pallas as pl
from jax.experimental.pallas import tpu as pltpu
from jax.experimental.pallas import tpu_sc as plsc

B = 128
P = 1024
D = 768
C = 512
K = 32
N = P * C

NC = 2
NS = 16
NW = NC * NS
ROWS_PER_W = B // NW

CH = 32768
NCH = N // CH
SUB = 64
SUBE = SUB * 16
NSUB = CH // SUBE
CAP = 8192

_NEG = float("-inf")
_POS = float("inf")
_BIG = 2 ** 30


def _sim_kernel(x_ref, t_ref, o_ref, cmax_ref, cmin_ref):
    t = t_ref[...]
    nrm = jnp.sqrt(jnp.sum(t * t, axis=1, keepdims=True)) + 1e-8
    tn = t / nrm
    s = jax.lax.dot_general(
        x_ref[0], tn, (((1,), (1,)), ((), ())),
        preferred_element_type=jnp.float32)
    o_ref[0] = s
    cmax_ref[0, 0, :] = jnp.max(s, axis=0)
    cmin_ref[0, 0, :] = jnp.min(s, axis=0)


def _thr_kernel(cmax_ref, cmin_ref, thi_ref, tlo_ref):
    def kth_largest(a):
        def body(_, a):
            m = jnp.max(a, axis=1, keepdims=True)
            return jnp.where(a == m, _NEG, a)
        a = lax.fori_loop(0, K - 1, body, a)
        return jnp.max(a, axis=1)

    thi = kth_largest(cmax_ref[...])
    tlo = -kth_largest(-cmin_ref[...])
    thi_ref[...] = jnp.broadcast_to(thi[:, None], (B, 128))
    tlo_ref[...] = jnp.broadcast_to(tlo[:, None], (B, 128))


def _bfly(v, lane, op):
    for sh in (8, 4, 2, 1):
        v = op(v, jnp.take(v, lane ^ sh))
    return v


def _sc_topk_kernel(sim_hbm, thi_hbm, tlo_hbm,
                    pv_hbm, pp_hbm, pc_hbm, nv_hbm, np_hbm, nc_hbm,
                    chunk_v, pvb, pib, nvb, nib, thi_v, tlo_v,
                    stage_f, stage_i, sem):
    wid = lax.axis_index("s") * NC + lax.axis_index("c")
    lane = lax.iota(jnp.int32, 16)

    def do_row(r, _):
        row = wid * ROWS_PER_W + r
        pltpu.sync_copy(thi_hbm.at[row, pl.ds(0, 16)], thi_v)
        pltpu.sync_copy(tlo_hbm.at[row, pl.ds(0, 16)], tlo_v)
        thi = thi_v[...]
        tlo = tlo_v[...]
        thi_s = thi[0]
        tlo_s = tlo[0]

        pltpu.make_async_copy(
            sim_hbm.at[row, pl.ds(0, CH)], chunk_v.at[pl.ds(0, CH)], sem
        ).start()

        def chunk_loop(c, carry):
            @pl.when(c + 1 < NCH)
            def _():
                pltpu.make_async_copy(
                    sim_hbm.at[row, pl.ds((c + 1) * CH, CH)],
                    chunk_v.at[pl.ds(((c + 1) % 2) * CH, CH)], sem
                ).start()
            cbase = (c % 2) * CH
            pltpu.make_async_copy(
                sim_hbm.at[row, pl.ds(c * CH, CH)],
                chunk_v.at[pl.ds(cbase, CH)], sem
            ).wait()

            def sub_body(s, carry2):
                base = cbase + s * SUBE
                LANES = 8
                amaxs = [jnp.full((16,), _NEG, jnp.float32)] * LANES
                amins = [jnp.full((16,), _POS, jnp.float32)] * LANES
                for u in range(SUB):
                    v = chunk_v[pl.ds(base + u * 16, 16)]
                    a = u % LANES
                    amaxs[a] = jnp.maximum(amaxs[a], v)
                    amins[a] = jnp.minimum(amins[a], v)
                amax, amin = amaxs[0], amins[0]
                for a in range(1, LANES):
                    amax = jnp.maximum(amax, amaxs[a])
                    amin = jnp.minimum(amin, amins[a])
                bmax = _bfly(amax, lane, jnp.maximum)[0]
                bmin = _bfly(amin, lane, jnp.minimum)[0]
                hit = (bmax >= thi_s) | (bmin <= tlo_s)

                def rescan(cr):
                    wp, wn = cr
                    fb = c * CH + s * SUBE
                    for u in range(SUB):
                        v = chunk_v[pl.ds(base + u * 16, 16)]
                        idx = lane + (fb + u * 16)
                        vm = _bfly(v, lane, jnp.maximum)[0]
                        vn = _bfly(v, lane, jnp.minimum)[0]
                        hp = (vm >= thi_s).astype(jnp.int32)
                        hn = (vn <= tlo_s).astype(jnp.int32)
                        h = v >= thi
                        l = v <= tlo
                        pvb[pl.ds(wp, 16)] = jnp.where(h, v, _NEG)
                        pib[pl.ds(wp, 16)] = jnp.where(h, idx, 0)
                        wp = jnp.minimum(wp + 16 * hp, CAP)
                        nvb[pl.ds(wn, 16)] = jnp.where(l, -v, _NEG)
                        nib[pl.ds(wn, 16)] = jnp.where(l, idx, 0)
                        wn = jnp.minimum(wn + 16 * hn, CAP)
                    return wp, wn

                return lax.cond(hit, rescan, lambda cr: cr, carry2)

            return lax.fori_loop(0, NSUB, sub_body, carry)

        wp, wn = lax.fori_loop(0, NCH, chunk_loop,
                               (jnp.int32(0), jnp.int32(0)))

        def extract(vbuf, ibuf, wcount, ov_hbm, op_hbm, oc_hbm):
            nvec = wcount // 16

            def kstep(k, carry):
                rv0, rv1, ri0, ri1 = carry

                def scan(i, c2):
                    bv, bi = c2
                    v = vbuf[pl.ds(i * 16, 16)]
                    ii = ibuf[pl.ds(i * 16, 16)]
                    m = v > bv
                    return jnp.where(m, v, bv), jnp.where(m, ii, bi)

                bv, bi = lax.fori_loop(
                    0, nvec, scan,
                    (jnp.full((16,), _NEG, jnp.float32),
                     jnp.zeros((16,), jnp.int32)))
                msp = _bfly(bv, lane, jnp.maximum)
                ssp = _bfly(jnp.where(bv == msp, bi, _BIG),
                            lane, jnp.minimum)

                def rem(i, _):
                    ii = ibuf[pl.ds(i * 16, 16)]
                    v = vbuf[pl.ds(i * 16, 16)]
                    vbuf[pl.ds(i * 16, 16)] = jnp.where(
                        ii == ssp, _NEG, v)
                    return 0
                lax.fori_loop(0, nvec, rem, 0)

                m0 = lane == k
                m1 = lane == (k - 16)
                rv0 = jnp.where(m0, msp, rv0)
                ri0 = jnp.where(m0, ssp, ri0)
                rv1 = jnp.where(m1, msp, rv1)
                ri1 = jnp.where(m1, ssp, ri1)
                return rv0, rv1, ri0, ri1

            z16f = jnp.zeros((16,), jnp.float32)
            z16i = jnp.zeros((16,), jnp.int32)
            rv0, rv1, ri0, ri1 = lax.fori_loop(
                0, K, kstep, (z16f, z16f, z16i, z16i))

            stage_f[pl.ds(0, 16)] = rv0
            stage_f[pl.ds(16, 16)] = rv1
            pltpu.sync_copy(stage_f, ov_hbm.at[row])
            nine = jnp.full((16,), 9, jnp.int32)
            stage_i[pl.ds(0, 16)] = lax.shift_right_logical(ri0, nine)
            stage_i[pl.ds(16, 16)] = lax.shift_right_logical(ri1, nine)
            pltpu.sync_copy(stage_i, op_hbm.at[row])
            cmask = jnp.full((16,), C - 1, jnp.int32)
            stage_i[pl.ds(0, 16)] = ri0 & cmask
            stage_i[pl.ds(16, 16)] = ri1 & cmask
            pltpu.sync_copy(stage_i, oc_hbm.at[row])

        extract(pvb, pib, wp, pv_hbm, pp_hbm, pc_hbm)
        extract(nvb, nib, wn, nv_hbm, np_hbm, nc_hbm)
        return 0

    lax.fori_loop(0, ROWS_PER_W, do_row, 0)


def kernel(test_features, feats_templates):
    sim, thi, tlo = pl.pallas_call(
        _sim_kernel,
        grid=(B,),
        in_specs=[
            pl.BlockSpec((1, P, D), lambda b: (b, 0, 0)),
            pl.BlockSpec((C, D), lambda b: (0, 0)),
        ],
        out_specs=[
            pl.BlockSpec((1, P, C), lambda b: (b, 0, 0)),
            pl.BlockSpec((1, 1, C), lambda b: (b, 0, 0)),
            pl.BlockSpec((1, 1, C), lambda b: (b, 0, 0)),
        ],
        out_shape=[
            jax.ShapeDtypeStruct((B, P, C), jnp.float32),
            jax.ShapeDtypeStruct((B, 1, C), jnp.float32),
            jax.ShapeDtypeStruct((B, 1, C), jnp.float32),
        ],
    )(test_features, feats_templates)
    cmax, cmin = thi, tlo

    thi2, tlo2 = pl.pallas_call(
        _thr_kernel,
        out_shape=[
            jax.ShapeDtypeStruct((B, 128), jnp.float32),
            jax.ShapeDtypeStruct((B, 128), jnp.float32),
        ],
    )(cmax.reshape(B, C), cmin.reshape(B, C))

    sim2 = sim.reshape(B, N)

    mesh = plsc.VectorSubcoreMesh(core_axis_name="c", subcore_axis_name="s")
    outs = [
        jax.ShapeDtypeStruct((B, K), jnp.float32),
        jax.ShapeDtypeStruct((B, K), jnp.int32),
        jax.ShapeDtypeStruct((B, K), jnp.int32),
        jax.ShapeDtypeStruct((B, K), jnp.float32),
        jax.ShapeDtypeStruct((B, K), jnp.int32),
        jax.ShapeDtypeStruct((B, K), jnp.int32),
    ]
    scratch = [
        pltpu.VMEM((2 * CH,), jnp.float32),
        pltpu.VMEM((CAP + 16,), jnp.float32),
        pltpu.VMEM((CAP + 16,), jnp.int32),
        pltpu.VMEM((CAP + 16,), jnp.float32),
        pltpu.VMEM((CAP + 16,), jnp.int32),
        pltpu.VMEM((16,), jnp.float32),
        pltpu.VMEM((16,), jnp.float32),
        pltpu.VMEM((K,), jnp.float32),
        pltpu.VMEM((K,), jnp.int32),
        pltpu.SemaphoreType.DMA,
    ]
    topk = pl.kernel(
        _sc_topk_kernel,
        out_type=outs,
        mesh=mesh,
        scratch_types=scratch,
    )
    return tuple(topk(sim2, thi2, tlo2))


# speedup vs baseline: 1.2345x; 1.2345x over previous
import jax
import jax.numpy as jnp
from jax import lax
from jax.experimental import pallas as pl
from jax.experimental.pallas import tpu as pltpu
from jax.experimental.pallas import tpu_sc as plsc

B = 128
P = 1024
D = 768
C = 512
K = 32
N = P * C

NC = 2
NS = 16
NW = NC * NS
ROWS_PER_W = B // NW

CH = 32768
NCH = N // CH
SUB = 64
SUBE = SUB * 16
NSUB = CH // SUBE
CAP = 8192

_NEG = float("-inf")
_POS = float("inf")
_BIG = 2 ** 30


def _sim_kernel(x_ref, t_ref, o_ref, cmax_ref, cmin_ref):
    t = t_ref[...]
    nrm = jnp.sqrt(jnp.sum(t * t, axis=1, keepdims=True)) + 1e-8
    tn = t / nrm
    s = jax.lax.dot_general(
        x_ref[0], tn, (((1,), (1,)), ((), ())),
        preferred_element_type=jnp.float32)
    o_ref[0] = s
    cmax_ref[0, 0, :] = jnp.max(s, axis=0)
    cmin_ref[0, 0, :] = jnp.min(s, axis=0)


def _thr_kernel(cmax_ref, cmin_ref, thi_ref, tlo_ref):
    def kth_largest(a):
        def body(_, a):
            m = jnp.max(a, axis=1, keepdims=True)
            return jnp.where(a == m, _NEG, a)
        a = lax.fori_loop(0, K - 1, body, a)
        return jnp.max(a, axis=1)

    thi = kth_largest(cmax_ref[...])
    tlo = -kth_largest(-cmin_ref[...])
    thi_ref[...] = jnp.broadcast_to(thi[:, None], (B, 128))
    tlo_ref[...] = jnp.broadcast_to(tlo[:, None], (B, 128))


def _bfly(v, lane, op):
    for sh in (8, 4, 2, 1):
        v = op(v, jnp.take(v, lane ^ sh))
    return v


def _sc_topk_kernel(sim_hbm, thi_hbm, tlo_hbm,
                    pv_hbm, pp_hbm, pc_hbm, nv_hbm, np_hbm, nc_hbm,
                    chunk_v, pvb, pib, nvb, nib, thi_v, tlo_v,
                    stage_f, stage_i, sem):
    wid = lax.axis_index("s") * NC + lax.axis_index("c")
    lane = lax.iota(jnp.int32, 16)

    def do_row(r, _):
        row = wid * ROWS_PER_W + r
        pltpu.sync_copy(thi_hbm.at[row, pl.ds(0, 16)], thi_v)
        pltpu.sync_copy(tlo_hbm.at[row, pl.ds(0, 16)], tlo_v)
        thi = thi_v[...]
        tlo = tlo_v[...]
        thi_s = thi[0]
        tlo_s = tlo[0]

        PCH = CH // C
        PSUB = SUBE // C
        pltpu.make_async_copy(
            sim_hbm.at[row, pl.ds(0, PCH), :], chunk_v.at[pl.ds(0, PCH), :], sem
        ).start()

        def chunk_loop(c, carry):
            @pl.when(c + 1 < NCH)
            def _():
                pltpu.make_async_copy(
                    sim_hbm.at[row, pl.ds((c + 1) * PCH, PCH), :],
                    chunk_v.at[pl.ds(((c + 1) % 2) * PCH, PCH), :], sem
                ).start()
            pbase = (c % 2) * PCH
            pltpu.make_async_copy(
                sim_hbm.at[row, pl.ds(c * PCH, PCH), :],
                chunk_v.at[pl.ds(pbase, PCH), :], sem
            ).wait()

            def sub_body(s, carry2):
                p0 = pbase + s * PSUB
                LANES = 8
                amaxs = [jnp.full((16,), _NEG, jnp.float32)] * LANES
                amins = [jnp.full((16,), _POS, jnp.float32)] * LANES
                for u in range(SUB):
                    v = chunk_v[p0 + u // 32, pl.ds((u % 32) * 16, 16)]
                    a = u % LANES
                    amaxs[a] = jnp.maximum(amaxs[a], v)
                    amins[a] = jnp.minimum(amins[a], v)
                amax, amin = amaxs[0], amins[0]
                for a in range(1, LANES):
                    amax = jnp.maximum(amax, amaxs[a])
                    amin = jnp.minimum(amin, amins[a])
                bmax = _bfly(amax, lane, jnp.maximum)[0]
                bmin = _bfly(amin, lane, jnp.minimum)[0]
                hit = (bmax >= thi_s) | (bmin <= tlo_s)

                def rescan(cr):
                    wp, wn = cr
                    fb = c * CH + s * SUBE
                    for u in range(SUB):
                        v = chunk_v[p0 + u // 32, pl.ds((u % 32) * 16, 16)]
                        idx = lane + (fb + u * 16)
                        vm = _bfly(v, lane, jnp.maximum)[0]
                        vn = _bfly(v, lane, jnp.minimum)[0]
                        hp = (vm >= thi_s).astype(jnp.int32)
                        hn = (vn <= tlo_s).astype(jnp.int32)
                        h = v >= thi
                        l = v <= tlo
                        pvb[pl.ds(wp, 16)] = jnp.where(h, v, _NEG)
                        pib[pl.ds(wp, 16)] = jnp.where(h, idx, 0)
                        wp = jnp.minimum(wp + 16 * hp, CAP)
                        nvb[pl.ds(wn, 16)] = jnp.where(l, -v, _NEG)
                        nib[pl.ds(wn, 16)] = jnp.where(l, idx, 0)
                        wn = jnp.minimum(wn + 16 * hn, CAP)
                    return wp, wn

                return lax.cond(hit, rescan, lambda cr: cr, carry2)

            return lax.fori_loop(0, NSUB, sub_body, carry)

        wp, wn = lax.fori_loop(0, NCH, chunk_loop,
                               (jnp.int32(0), jnp.int32(0)))

        def extract(vbuf, ibuf, wcount, ov_hbm, op_hbm, oc_hbm):
            nvec = wcount // 16

            def kstep(k, carry):
                rv0, rv1, ri0, ri1 = carry

                def scan(i, c2):
                    bv, bi = c2
                    v = vbuf[pl.ds(i * 16, 16)]
                    ii = ibuf[pl.ds(i * 16, 16)]
                    m = v > bv
                    return jnp.where(m, v, bv), jnp.where(m, ii, bi)

                bv, bi = lax.fori_loop(
                    0, nvec, scan,
                    (jnp.full((16,), _NEG, jnp.float32),
                     jnp.zeros((16,), jnp.int32)))
                msp = _bfly(bv, lane, jnp.maximum)
                ssp = _bfly(jnp.where(bv == msp, bi, _BIG),
                            lane, jnp.minimum)

                def rem(i, _):
                    ii = ibuf[pl.ds(i * 16, 16)]
                    v = vbuf[pl.ds(i * 16, 16)]
                    vbuf[pl.ds(i * 16, 16)] = jnp.where(
                        ii == ssp, _NEG, v)
                    return 0
                lax.fori_loop(0, nvec, rem, 0)

                m0 = lane == k
                m1 = lane == (k - 16)
                rv0 = jnp.where(m0, msp, rv0)
                ri0 = jnp.where(m0, ssp, ri0)
                rv1 = jnp.where(m1, msp, rv1)
                ri1 = jnp.where(m1, ssp, ri1)
                return rv0, rv1, ri0, ri1

            z16f = jnp.zeros((16,), jnp.float32)
            z16i = jnp.zeros((16,), jnp.int32)
            rv0, rv1, ri0, ri1 = lax.fori_loop(
                0, K, kstep, (z16f, z16f, z16i, z16i))

            stage_f[pl.ds(0, 16)] = rv0
            stage_f[pl.ds(16, 16)] = rv1
            pltpu.sync_copy(stage_f, ov_hbm.at[row])
            nine = jnp.full((16,), 9, jnp.int32)
            stage_i[pl.ds(0, 16)] = lax.shift_right_logical(ri0, nine)
            stage_i[pl.ds(16, 16)] = lax.shift_right_logical(ri1, nine)
            pltpu.sync_copy(stage_i, op_hbm.at[row])
            cmask = jnp.full((16,), C - 1, jnp.int32)
            stage_i[pl.ds(0, 16)] = ri0 & cmask
            stage_i[pl.ds(16, 16)] = ri1 & cmask
            pltpu.sync_copy(stage_i, oc_hbm.at[row])

        extract(pvb, pib, wp, pv_hbm, pp_hbm, pc_hbm)
        extract(nvb, nib, wn, nv_hbm, np_hbm, nc_hbm)
        return 0

    lax.fori_loop(0, ROWS_PER_W, do_row, 0)


def kernel(test_features, feats_templates):
    sim, thi, tlo = pl.pallas_call(
        _sim_kernel,
        grid=(B,),
        in_specs=[
            pl.BlockSpec((1, P, D), lambda b: (b, 0, 0)),
            pl.BlockSpec((C, D), lambda b: (0, 0)),
        ],
        out_specs=[
            pl.BlockSpec((1, P, C), lambda b: (b, 0, 0)),
            pl.BlockSpec((1, 1, C), lambda b: (b, 0, 0)),
            pl.BlockSpec((1, 1, C), lambda b: (b, 0, 0)),
        ],
        out_shape=[
            jax.ShapeDtypeStruct((B, P, C), jnp.float32),
            jax.ShapeDtypeStruct((B, 1, C), jnp.float32),
            jax.ShapeDtypeStruct((B, 1, C), jnp.float32),
        ],
    )(test_features, feats_templates)
    cmax, cmin = thi, tlo

    thi2, tlo2 = pl.pallas_call(
        _thr_kernel,
        out_shape=[
            jax.ShapeDtypeStruct((B, 128), jnp.float32),
            jax.ShapeDtypeStruct((B, 128), jnp.float32),
        ],
    )(cmax.reshape(B, C), cmin.reshape(B, C))

    sim2 = sim

    mesh = plsc.VectorSubcoreMesh(core_axis_name="c", subcore_axis_name="s")
    outs = [
        jax.ShapeDtypeStruct((B, K), jnp.float32),
        jax.ShapeDtypeStruct((B, K), jnp.int32),
        jax.ShapeDtypeStruct((B, K), jnp.int32),
        jax.ShapeDtypeStruct((B, K), jnp.float32),
        jax.ShapeDtypeStruct((B, K), jnp.int32),
        jax.ShapeDtypeStruct((B, K), jnp.int32),
    ]
    scratch = [
        pltpu.VMEM((2 * (CH // C), C), jnp.float32),
        pltpu.VMEM((CAP + 16,), jnp.float32),
        pltpu.VMEM((CAP + 16,), jnp.int32),
        pltpu.VMEM((CAP + 16,), jnp.float32),
        pltpu.VMEM((CAP + 16,), jnp.int32),
        pltpu.VMEM((16,), jnp.float32),
        pltpu.VMEM((16,), jnp.float32),
        pltpu.VMEM((K,), jnp.float32),
        pltpu.VMEM((K,), jnp.int32),
        pltpu.SemaphoreType.DMA,
    ]
    topk = pl.kernel(
        _sc_topk_kernel,
        out_type=outs,
        mesh=mesh,
        scratch_types=scratch,
    )
    return tuple(topk(sim2, thi2, tlo2))
